# baseline (speedup 1.0000x reference)
import functools

import jax
import jax.numpy as jnp
from jax import lax
from jax.experimental import pallas as pl
from jax.experimental.pallas import tpu as pltpu
from jax.experimental.pallas import tpu_sc as plsc

SC_NC = 2
SC_NS = 16
SC_NW = SC_NC * SC_NS

S = 2048
D_MODEL = 1024
E = 8
D_FF = 4096
K = 2
N_ITEMS = S * K

M = 256
NB = 24
NPAD = NB * M
F_BLK = 512
NF = D_FF // F_BLK


def _router_body(x_ref, gw_ref, probs_ref, gates_ref, eidx_ref):
    x = x_ref[...]
    gw = gw_ref[...]
    logits = jax.lax.dot_general(
        x, gw, (((1,), (1,)), ((), ())), preferred_element_type=jnp.float32)
    m = jnp.max(logits, axis=-1, keepdims=True)
    ex = jnp.exp(logits - m)
    probs = ex / jnp.sum(ex, axis=-1, keepdims=True)
    probs_ref[...] = probs

    iota = jax.lax.broadcasted_iota(jnp.int32, probs.shape, 1)
    m1 = jnp.max(probs, axis=-1, keepdims=True)
    i1 = jnp.min(jnp.where(probs == m1, iota, E), axis=-1, keepdims=True)
    probs_lo = jnp.where(iota == i1, -1.0, probs)
    m2 = jnp.max(probs_lo, axis=-1, keepdims=True)
    i2 = jnp.min(jnp.where(probs_lo == m2, iota, E), axis=-1, keepdims=True)
    s = m1 + m2
    gates_ref[...] = jnp.concatenate([m1 / s, m2 / s], axis=1)
    eidx_ref[...] = jnp.concatenate([i1, i2], axis=1)


def _grouped_body(be_ref, na_ref, xs_ref, sg_ref, wi_ref, wv_ref, wo_ref,
                  ys_ref, wi16_s, wv16_s, wo16_s):
    f = pl.program_id(0)
    b = pl.program_id(1)
    rows = pl.ds(b * M, M)

    changed = jnp.logical_or(
        b == 0, be_ref[b] != be_ref[jnp.maximum(b - 1, 0)])

    @pl.when(jnp.logical_and(changed, b < na_ref[0]))
    def _():
        wi16_s[...] = wi_ref[0].astype(jnp.bfloat16)
        wv16_s[...] = wv_ref[0].astype(jnp.bfloat16)
        wo16_s[...] = wo_ref[0].astype(jnp.bfloat16)

    @pl.when(b < na_ref[0])
    def _():
        x16 = xs_ref[rows, :]
        a = jax.lax.dot_general(
            x16, wi16_s[...], (((1,), (1,)), ((), ())),
            preferred_element_type=jnp.float32)
        v = jax.lax.dot_general(
            x16, wv16_s[...], (((1,), (1,)), ((), ())),
            preferred_element_type=jnp.float32)
        g = 0.5 * a * (1.0 + jax.lax.erf(a * 0.7071067811865476))
        h = (g * v).astype(jnp.bfloat16)
        part = jax.lax.dot_general(
            h, wo16_s[...], (((1,), (1,)), ((), ())),
            preferred_element_type=jnp.float32)

        @pl.when(f == 0)
        def _():
            ys_ref[rows, :] = part

        @pl.when(jnp.logical_and(f != 0, f != NF - 1))
        def _():
            ys_ref[rows, :] += part

        @pl.when(f == NF - 1)
        def _():
            g = sg_ref[:, 0:1]
            ys_ref[rows, :] = (ys_ref[rows, :] + part) * g


def _routing_metadata(eidx):
    ef = eidx.reshape(-1).astype(jnp.int32)
    onehot = (ef[:, None] == jnp.arange(E, dtype=jnp.int32)[None, :])
    csum = jnp.cumsum(onehot.astype(jnp.int32), axis=0)
    counts = csum[-1]
    rank = jnp.take_along_axis(csum, ef[:, None], axis=1)[:, 0] - 1
    nblk = (counts + M - 1) // M
    cumblk = jnp.cumsum(nblk)
    n_active = cumblk[-1]
    bstart = (cumblk - nblk) * M
    pos = (bstart[ef] + rank).astype(jnp.int32)
    sorted_tok = (jnp.arange(NPAD, dtype=jnp.int32) % S).at[pos].set(
        jnp.arange(N_ITEMS, dtype=jnp.int32) // K)
    be = jnp.sum((jnp.arange(NB, dtype=jnp.int32)[:, None]
                  >= cumblk[None, :]).astype(jnp.int32), axis=1)
    last_e = jnp.max(jnp.where(nblk > 0, jnp.arange(E, dtype=jnp.int32), 0))
    be = jnp.where(jnp.arange(NB) < n_active, jnp.minimum(be, E - 1), last_e)
    return sorted_tok, pos, be, n_active.reshape(1)


RPW = NPAD // SC_NW
RCH = 96
TPW = S // SC_NW
XW = D_MODEL // 2

_sc_mesh = plsc.VectorSubcoreMesh(core_axis_name="c", subcore_axis_name="s")


def _dispatch_body(tok_hbm, xw_hbm, xs_hbm, idx_v, rows_v, sem):
    wid = lax.axis_index("s") * SC_NC + lax.axis_index("c")
    for c in range(RPW // RCH):
        base = wid * RPW + c * RCH
        pltpu.sync_copy(tok_hbm.at[pl.ds(base, RCH)], idx_v)
        pltpu.async_copy(xw_hbm.at[idx_v], rows_v, sem).wait()
        pltpu.sync_copy(rows_v, xs_hbm.at[pl.ds(base, RCH)])


_dispatch = pl.kernel(
    _dispatch_body,
    out_type=jax.ShapeDtypeStruct((NPAD, XW), jnp.int32),
    scratch_types=[
        pltpu.VMEM((RCH,), jnp.int32),
        pltpu.VMEM((RCH, XW), jnp.int32),
        pltpu.SemaphoreType.DMA,
    ],
    mesh=_sc_mesh,
)


def _combine_body(pos_hbm, ys_hbm, out_hbm, idx_v, rows_v, out_v, sem):
    wid = lax.axis_index("s") * SC_NC + lax.axis_index("c")
    for c in range(TPW // 16):
        tbase = wid * TPW + c * 16
        pltpu.sync_copy(pos_hbm.at[pl.ds(tbase * K, 16 * K)], idx_v)
        pltpu.async_copy(ys_hbm.at[idx_v], rows_v, sem).wait()

        def tok_body(i, carry):
            for d in range(D_MODEL // 16):
                sl = pl.ds(d * 16, 16)
                out_v[i, sl] = rows_v[2 * i, sl] + rows_v[2 * i + 1, sl]
            return carry

        lax.fori_loop(0, 16, tok_body, 0)
        pltpu.sync_copy(out_v, out_hbm.at[pl.ds(tbase, 16)])


_combine = pl.kernel(
    _combine_body,
    out_type=jax.ShapeDtypeStruct((S, D_MODEL), jnp.float32),
    scratch_types=[
        pltpu.VMEM((16 * K,), jnp.int32),
        pltpu.VMEM((16 * K, D_MODEL), jnp.float32),
        pltpu.VMEM((16, D_MODEL), jnp.float32),
        pltpu.SemaphoreType.DMA,
    ],
    mesh=_sc_mesh,
)


def kernel(x, gate_w, w_in, w_v, w_out):
    x2 = x.reshape(S, D_MODEL)

    probs, gates, eidx = pl.pallas_call(
        _router_body,
        out_shape=(
            jax.ShapeDtypeStruct((S, E), jnp.float32),
            jax.ShapeDtypeStruct((S, K), jnp.float32),
            jax.ShapeDtypeStruct((S, K), jnp.int32),
        ),
    )(x2, gate_w)

    sorted_tok, pos, be, n_active = _routing_metadata(eidx)

    sg = jnp.zeros((NPAD,), jnp.float32).at[pos].set(gates.reshape(-1))
    sg_b = jnp.broadcast_to(sg[:, None], (NPAD, 128))

    x16i = lax.bitcast_convert_type(
        x2.astype(jnp.bfloat16).reshape(S, XW, 2), jnp.int32)
    xs_i = _dispatch(sorted_tok, x16i)
    xs = lax.bitcast_convert_type(xs_i, jnp.bfloat16).reshape(NPAD, D_MODEL)

    ys = pl.pallas_call(
        _grouped_body,
        grid_spec=pltpu.PrefetchScalarGridSpec(
            num_scalar_prefetch=2,
            grid=(NF, NB),
            in_specs=[
                pl.BlockSpec((NPAD, D_MODEL),
                             lambda f, b, be_ref, na_ref: (0, 0)),
                pl.BlockSpec((M, 128),
                             lambda f, b, be_ref, na_ref: (b, 0)),
                pl.BlockSpec((1, F_BLK, D_MODEL),
                             lambda f, b, be_ref, na_ref: (be_ref[b], f, 0)),
                pl.BlockSpec((1, F_BLK, D_MODEL),
                             lambda f, b, be_ref, na_ref: (be_ref[b], f, 0)),
                pl.BlockSpec((1, D_MODEL, F_BLK),
                             lambda f, b, be_ref, na_ref: (be_ref[b], 0, f)),
            ],
            out_specs=pl.BlockSpec((NPAD, D_MODEL),
                                   lambda f, b, be_ref, na_ref: (0, 0)),
            scratch_shapes=[
                pltpu.VMEM((F_BLK, D_MODEL), jnp.bfloat16),
                pltpu.VMEM((F_BLK, D_MODEL), jnp.bfloat16),
                pltpu.VMEM((D_MODEL, F_BLK), jnp.bfloat16),
            ],
        ),
        out_shape=jax.ShapeDtypeStruct((NPAD, D_MODEL), jnp.float32),
        compiler_params=pltpu.CompilerParams(
            vmem_limit_bytes=100 * 1024 * 1024),
    )(be, n_active, xs, sg_b, w_in, w_v, w_out)

    out = _combine(pos, ys)

    return out.reshape(1, S, D_MODEL), probs.reshape(1, S, E)


# speedup vs baseline: 1.0112x; 1.0112x over previous
import functools

import jax
import jax.numpy as jnp
from jax import lax
from jax.experimental import pallas as pl
from jax.experimental.pallas import tpu as pltpu
from jax.experimental.pallas import tpu_sc as plsc

SC_NC = 2
SC_NS = 16
SC_NW = SC_NC * SC_NS

S = 2048
D_MODEL = 1024
E = 8
D_FF = 4096
K = 2
N_ITEMS = S * K

M = 256
NB = 24
NPAD = NB * M
F_BLK = 512
NF = D_FF // F_BLK


def _router_body(x_ref, gw_ref, probs_ref, gates_ref, eidx_ref):
    x = x_ref[...]
    gw = gw_ref[...]
    logits = jax.lax.dot_general(
        x, gw, (((1,), (1,)), ((), ())), preferred_element_type=jnp.float32)
    m = jnp.max(logits, axis=-1, keepdims=True)
    ex = jnp.exp(logits - m)
    probs = ex / jnp.sum(ex, axis=-1, keepdims=True)
    probs_ref[...] = probs

    iota = jax.lax.broadcasted_iota(jnp.int32, probs.shape, 1)
    m1 = jnp.max(probs, axis=-1, keepdims=True)
    i1 = jnp.min(jnp.where(probs == m1, iota, E), axis=-1, keepdims=True)
    probs_lo = jnp.where(iota == i1, -1.0, probs)
    m2 = jnp.max(probs_lo, axis=-1, keepdims=True)
    i2 = jnp.min(jnp.where(probs_lo == m2, iota, E), axis=-1, keepdims=True)
    s = m1 + m2
    gates_ref[...] = jnp.concatenate([m1 / s, m2 / s], axis=1)
    eidx_ref[...] = jnp.concatenate([i1, i2], axis=1)


def _grouped_body(be_ref, na_ref, xs_ref, sg_ref, wi_ref, wv_ref, wo_ref,
                  ys_ref, wi16_s, wv16_s, wo16_s):
    f = pl.program_id(0)
    b = pl.program_id(1)
    rows = pl.ds(b * M, M)

    changed = jnp.logical_or(
        b == 0, be_ref[b] != be_ref[jnp.maximum(b - 1, 0)])

    @pl.when(jnp.logical_and(changed, b < na_ref[0]))
    def _():
        wi16_s[...] = wi_ref[0].astype(jnp.bfloat16)
        wv16_s[...] = wv_ref[0].astype(jnp.bfloat16)
        wo16_s[...] = wo_ref[0].astype(jnp.bfloat16)

    @pl.when(b < na_ref[0])
    def _():
        x16 = xs_ref[rows, :]
        a = jax.lax.dot_general(
            x16, wi16_s[...], (((1,), (1,)), ((), ())),
            preferred_element_type=jnp.float32)
        v = jax.lax.dot_general(
            x16, wv16_s[...], (((1,), (1,)), ((), ())),
            preferred_element_type=jnp.float32)
        g = 0.5 * a * (1.0 + jax.lax.erf(a * 0.7071067811865476))
        h = (g * v).astype(jnp.bfloat16)
        part = jax.lax.dot_general(
            h, wo16_s[...], (((1,), (1,)), ((), ())),
            preferred_element_type=jnp.float32)

        @pl.when(f == 0)
        def _():
            ys_ref[rows, :] = part

        @pl.when(jnp.logical_and(f != 0, f != NF - 1))
        def _():
            ys_ref[rows, :] += part

        @pl.when(f == NF - 1)
        def _():
            g = sg_ref[:, 0:1]
            ys_ref[rows, :] = (ys_ref[rows, :] + part) * g


def _routing_metadata(eidx):
    ef = eidx.reshape(-1).astype(jnp.int32)
    onehot = (ef[:, None] == jnp.arange(E, dtype=jnp.int32)[None, :])
    csum = jnp.cumsum(onehot.astype(jnp.int32), axis=0)
    counts = csum[-1]
    rank = jnp.take_along_axis(csum, ef[:, None], axis=1)[:, 0] - 1
    nblk = (counts + M - 1) // M
    cumblk = jnp.cumsum(nblk)
    n_active = cumblk[-1]
    bstart = (cumblk - nblk) * M
    pos = (bstart[ef] + rank).astype(jnp.int32)
    sorted_tok = (jnp.arange(NPAD, dtype=jnp.int32) % S).at[pos].set(
        jnp.arange(N_ITEMS, dtype=jnp.int32) // K)
    be = jnp.sum((jnp.arange(NB, dtype=jnp.int32)[:, None]
                  >= cumblk[None, :]).astype(jnp.int32), axis=1)
    last_e = jnp.max(jnp.where(nblk > 0, jnp.arange(E, dtype=jnp.int32), 0))
    be = jnp.where(jnp.arange(NB) < n_active, jnp.minimum(be, E - 1), last_e)
    return sorted_tok, pos, be, n_active.reshape(1)


RPW = NPAD // SC_NW
RCH = 96
TPW = S // SC_NW
XW = D_MODEL // 2

_sc_mesh = plsc.VectorSubcoreMesh(core_axis_name="c", subcore_axis_name="s")


def _dispatch_body(tok_hbm, xw_hbm, xs_hbm, idx_v, rows_v, sem):
    wid = lax.axis_index("s") * SC_NC + lax.axis_index("c")
    for c in range(RPW // RCH):
        base = wid * RPW + c * RCH
        pltpu.sync_copy(tok_hbm.at[pl.ds(base, RCH)], idx_v)
        pltpu.async_copy(xw_hbm.at[idx_v], rows_v, sem).wait()
        pltpu.sync_copy(rows_v, xs_hbm.at[pl.ds(base, RCH)])


_dispatch = pl.kernel(
    _dispatch_body,
    out_type=jax.ShapeDtypeStruct((NPAD, XW), jnp.int32),
    scratch_types=[
        pltpu.VMEM((RCH,), jnp.int32),
        pltpu.VMEM((RCH, XW), jnp.int32),
        pltpu.SemaphoreType.DMA,
    ],
    mesh=_sc_mesh,
)


def _combine_body(pos_hbm, ys_hbm, out_hbm, idx_v, rows_v, out_v, sem):
    wid = lax.axis_index("s") * SC_NC + lax.axis_index("c")
    nch = TPW // 16

    def fetch(c, buf):
        tbase = wid * TPW + c * 16
        pltpu.sync_copy(pos_hbm.at[pl.ds(tbase * K, 16 * K)],
                        idx_v.at[buf])
        pltpu.async_copy(ys_hbm.at[idx_v.at[buf]], rows_v.at[buf], sem)

    fetch(0, 0)
    for c in range(nch):
        buf = c % 2
        if c + 1 < nch:
            fetch(c + 1, (c + 1) % 2)
        pltpu.make_async_copy(
            ys_hbm.at[idx_v.at[buf]], rows_v.at[buf], sem).wait()
        tbase = wid * TPW + c * 16

        def tok_body(i, carry):
            for d in range(D_MODEL // 16):
                sl = pl.ds(d * 16, 16)
                out_v[i, sl] = (rows_v[buf, 2 * i, sl]
                                + rows_v[buf, 2 * i + 1, sl])
            return carry

        lax.fori_loop(0, 16, tok_body, 0)
        pltpu.sync_copy(out_v, out_hbm.at[pl.ds(tbase, 16)])


_combine = pl.kernel(
    _combine_body,
    out_type=jax.ShapeDtypeStruct((S, D_MODEL), jnp.float32),
    scratch_types=[
        pltpu.VMEM((2, 16 * K), jnp.int32),
        pltpu.VMEM((2, 16 * K, D_MODEL), jnp.float32),
        pltpu.VMEM((16, D_MODEL), jnp.float32),
        pltpu.SemaphoreType.DMA,
    ],
    mesh=_sc_mesh,
)


def kernel(x, gate_w, w_in, w_v, w_out):
    x2 = x.reshape(S, D_MODEL)

    probs, gates, eidx = pl.pallas_call(
        _router_body,
        out_shape=(
            jax.ShapeDtypeStruct((S, E), jnp.float32),
            jax.ShapeDtypeStruct((S, K), jnp.float32),
            jax.ShapeDtypeStruct((S, K), jnp.int32),
        ),
    )(x2, gate_w)

    sorted_tok, pos, be, n_active = _routing_metadata(eidx)

    sg = jnp.zeros((NPAD,), jnp.float32).at[pos].set(gates.reshape(-1))
    sg_b = jnp.broadcast_to(sg[:, None], (NPAD, 128))

    x16i = lax.bitcast_convert_type(
        x2.astype(jnp.bfloat16).reshape(S, XW, 2), jnp.int32)
    xs_i = _dispatch(sorted_tok, x16i)
    xs = lax.bitcast_convert_type(xs_i, jnp.bfloat16).reshape(NPAD, D_MODEL)

    ys = pl.pallas_call(
        _grouped_body,
        grid_spec=pltpu.PrefetchScalarGridSpec(
            num_scalar_prefetch=2,
            grid=(NF, NB),
            in_specs=[
                pl.BlockSpec((NPAD, D_MODEL),
                             lambda f, b, be_ref, na_ref: (0, 0)),
                pl.BlockSpec((M, 128),
                             lambda f, b, be_ref, na_ref: (b, 0)),
                pl.BlockSpec((1, F_BLK, D_MODEL),
                             lambda f, b, be_ref, na_ref: (be_ref[b], f, 0)),
                pl.BlockSpec((1, F_BLK, D_MODEL),
                             lambda f, b, be_ref, na_ref: (be_ref[b], f, 0)),
                pl.BlockSpec((1, D_MODEL, F_BLK),
                             lambda f, b, be_ref, na_ref: (be_ref[b], 0, f)),
            ],
            out_specs=pl.BlockSpec((NPAD, D_MODEL),
                                   lambda f, b, be_ref, na_ref: (0, 0)),
            scratch_shapes=[
                pltpu.VMEM((F_BLK, D_MODEL), jnp.bfloat16),
                pltpu.VMEM((F_BLK, D_MODEL), jnp.bfloat16),
                pltpu.VMEM((D_MODEL, F_BLK), jnp.bfloat16),
            ],
        ),
        out_shape=jax.ShapeDtypeStruct((NPAD, D_MODEL), jnp.float32),
        compiler_params=pltpu.CompilerParams(
            vmem_limit_bytes=100 * 1024 * 1024),
    )(be, n_active, xs, sg_b, w_in, w_v, w_out)

    out = _combine(pos, ys)

    return out.reshape(1, S, D_MODEL), probs.reshape(1, S, E)
